# baseline (speedup 1.0000x reference)
import functools

import jax
import jax.numpy as jnp
from jax import lax
from jax.experimental import pallas as pl
from jax.experimental.pallas import tpu as pltpu

NEG = float("-inf")
IBIG = 2 ** 30


def _topk_body(q_ref, m_ref, vals_out, idx_out, vs, is_, qn_bf, work, mxv,
               *, kt, nk, kreal):
    k = pl.program_id(0)
    n_q = qn_bf.shape[0]

    @pl.when(k == 0)
    def _():
        vs[...] = jnp.full(vs.shape, NEG, jnp.float32)
        is_[...] = jnp.full(is_.shape, IBIG, jnp.int32)
        q_blk = q_ref[...]
        q2 = jnp.sum(q_blk * q_blk, axis=1, keepdims=True)
        qn = q_blk / (jnp.sqrt(q2) + 1e-6)
        qn_bf[...] = qn.astype(jnp.bfloat16)

    m = m_ref[...]
    ss = jnp.sum(m * m, axis=1, keepdims=True)
    mn = (m / (jnp.sqrt(ss) + 1e-6)).astype(jnp.bfloat16)
    sims = lax.dot_general(mn, qn_bf[...], (((1,), (1,)), ((), ())),
                           preferred_element_type=jnp.float32)
    cb = k * kt + lax.broadcasted_iota(jnp.int32, (kt, 1), 0)
    work[...] = sims

    @pl.when(k == nk - 1)
    def _():
        if nk * kt != kreal:
            work[...] = jnp.where(cb < kreal, work[...], NEG)

    mxv[0:1, :] = jnp.max(work[...], axis=0, keepdims=True)

    def _cond(go):
        return go

    def _body(_):
        mx = mxv[0:1, :]
        t5 = vs[4:5, :]
        upd = mx > t5
        go = jnp.any(upd)

        @pl.when(go)
        def _():
            w = work[...]
            sel = jnp.min(jnp.where(w == mx, cb, IBIG), axis=0,
                          keepdims=True)
            v8 = vs[...]
            i8 = is_[...]
            ge = v8 >= mx
            gef = ge.astype(jnp.float32)
            gesh = jnp.concatenate(
                [jnp.ones((1, n_q), jnp.float32), gef[:7]], axis=0) > 0.5
            vsh = jnp.concatenate([v8[:1], v8[:7]], axis=0)
            ish = jnp.concatenate([i8[:1], i8[:7]], axis=0)
            nv = jnp.where(ge, v8, jnp.where(gesh, mx, vsh))
            ni = jnp.where(ge, i8, jnp.where(gesh, sel, ish))
            vs[...] = jnp.where(upd, nv, v8)
            is_[...] = jnp.where(upd, ni, i8)
            w2 = jnp.where(cb == sel, NEG, w)
            work[...] = w2
            mxv[0:1, :] = jnp.max(w2, axis=0, keepdims=True)

        return go

    lax.while_loop(_cond, _body, True)

    @pl.when(k == nk - 1)
    def _():
        vals_out[...] = vs[...]
        idx_out[...] = is_[...]


def _topk_pallas(q, memory, *, interpret=False):
    qn, d = q.shape
    kreal = memory.shape[0]
    kt = 256
    nk = pl.cdiv(kreal, kt)
    body = functools.partial(_topk_body, kt=kt, nk=nk, kreal=kreal)
    return pl.pallas_call(
        body,
        grid=(nk,),
        in_specs=[
            pl.BlockSpec((qn, d), lambda k: (0, 0)),
            pl.BlockSpec((kt, d), lambda k: (k, 0)),
        ],
        out_specs=[
            pl.BlockSpec((8, qn), lambda k: (0, 0)),
            pl.BlockSpec((8, qn), lambda k: (0, 0)),
        ],
        out_shape=[
            jax.ShapeDtypeStruct((8, qn), jnp.float32),
            jax.ShapeDtypeStruct((8, qn), jnp.int32),
        ],
        scratch_shapes=[
            pltpu.VMEM((8, qn), jnp.float32),
            pltpu.VMEM((8, qn), jnp.int32),
            pltpu.VMEM((qn, d), jnp.bfloat16),
            pltpu.VMEM((kt, qn), jnp.float32),
            pltpu.VMEM((8, qn), jnp.float32),
        ],
        compiler_params=pltpu.CompilerParams(
            dimension_semantics=("arbitrary",),
        ),
        interpret=interpret,
    )(q, memory)


def _sc_gather(memory, idx3):
    from jax.experimental.pallas import tpu_sc as plsc

    nw, n_chunks, cw = idx3.shape
    d = memory.shape[1]
    n_rows = nw * n_chunks * cw
    mesh = plsc.VectorSubcoreMesh(core_axis_name="c", subcore_axis_name="s")
    info = plsc.get_sparse_core_info()
    nc = info.num_cores

    @functools.partial(
        pl.kernel,
        mesh=mesh,
        out_type=jax.ShapeDtypeStruct((n_rows, d), jnp.float32),
        scratch_types=[
            pltpu.VMEM((n_chunks, cw), jnp.int32),
            pltpu.VMEM((cw, d), jnp.float32),
            pltpu.SemaphoreType.DMA,
        ],
    )
    def gather_k(mem_hbm, idx_hbm, out_hbm, idx_v, rows_v, sem):
        wid = lax.axis_index("s") * nc + lax.axis_index("c")
        pltpu.sync_copy(idx_hbm.at[wid], idx_v)
        base = wid * (n_chunks * cw)
        for ch in range(n_chunks):
            pltpu.async_copy(mem_hbm.at[idx_v.at[ch]], rows_v, sem).wait()
            pltpu.sync_copy(rows_v, out_hbm.at[pl.ds(base + ch * cw, cw)])

    return gather_k(memory, idx3)


def _final_body(x_ref, neigh_ref, vals_ref, scores_ref, out_ref, loss_ref,
                *, n_tok):
    i = pl.program_id(0)
    x = x_ref[...]
    q2 = jnp.sum(x * x, axis=-1, keepdims=True)
    vals = vals_ref[...]
    w = jax.nn.softmax(vals, axis=-1)
    neigh = neigh_ref[...]
    retrieved = jnp.sum(w[..., None] * neigh, axis=2)
    mem_emb = 0.1 * x + 0.9 * retrieved
    out_ref[...] = mem_emb
    num = jnp.sum(x * mem_emb, axis=-1)
    den = (jnp.sqrt(q2[..., 0])
           * jnp.sqrt(jnp.sum(mem_emb * mem_emb, axis=-1)) + 1e-6)
    cos = num / den
    s = scores_ref[0]
    sn = s / (jnp.sum(s, axis=1, keepdims=True) + 1e-6)
    contrib = jnp.sum((1.0 - cos) * sn) / n_tok

    @pl.when(i == 0)
    def _():
        loss_ref[0, 0] = contrib

    @pl.when(i != 0)
    def _():
        loss_ref[0, 0] = loss_ref[0, 0] + contrib


def _final_pallas(x, neigh, vals8, scores, *, interpret=False):
    b, n, d = x.shape
    bb = 2
    grid = (b // bb,)
    body = functools.partial(_final_body, n_tok=float(b * n))
    return pl.pallas_call(
        body,
        grid=grid,
        in_specs=[
            pl.BlockSpec((bb, n, d), lambda i: (i, 0, 0)),
            pl.BlockSpec((bb, n, 5, d), lambda i: (i, 0, 0, 0)),
            pl.BlockSpec((bb, n, 5), lambda i: (i, 0, 0)),
            pl.BlockSpec((1, bb, n), lambda i: (i, 0, 0)),
        ],
        out_specs=[
            pl.BlockSpec((bb, n, d), lambda i: (i, 0, 0)),
            pl.BlockSpec((1, 1), lambda i: (0, 0),
                         memory_space=pltpu.SMEM),
        ],
        out_shape=[
            jax.ShapeDtypeStruct((b, n, d), jnp.float32),
            jax.ShapeDtypeStruct((1, 1), jnp.float32),
        ],
        compiler_params=pltpu.CompilerParams(
            dimension_semantics=("arbitrary",),
        ),
        interpret=interpret,
    )(x, neigh, vals8, scores.reshape(b // bb, bb, n))


def _run(x, memory, combined_scores, *, interpret=False, gather_fn=None):
    b, n, d = x.shape
    q = x.reshape(b * n, d)
    nq = b * n
    vals8t, idx8t = _topk_pallas(q, memory, interpret=interpret)
    idx_flat = idx8t[:5].T.reshape(nq * 5)
    n_chunks = pl.cdiv(nq * 5, 32 * 128)
    n_pad = 32 * n_chunks * 128
    idx_pad = jnp.concatenate(
        [idx_flat, jnp.zeros((n_pad - nq * 5,), jnp.int32)])
    idx3 = idx_pad.reshape(32, n_chunks, 128)
    if gather_fn is None:
        neigh_flat = _sc_gather(memory, idx3)
    else:
        neigh_flat = gather_fn(memory, idx3)
    neigh = neigh_flat[:nq * 5].reshape(b, n, 5, d)
    vals5 = vals8t[:5].T.reshape(b, n, 5)
    mem_emb, loss = _final_pallas(x, neigh, vals5, combined_scores,
                                  interpret=interpret)
    return mem_emb, loss[0, 0]


def kernel(x, memory, combined_scores, num_neighbors):
    del num_neighbors
    return _run(x, memory, combined_scores)


# speedup vs baseline: 1.0102x; 1.0102x over previous
import functools

import jax
import jax.numpy as jnp
from jax import lax
from jax.experimental import pallas as pl
from jax.experimental.pallas import tpu as pltpu

NEG = float("-inf")
IBIG = 2 ** 30


def _topk_body(q_ref, m_ref, vals_out, idx_out, vs, is_, qn_bf, work, mxv,
               *, kt, nk, kreal):
    k = pl.program_id(0)
    n_q = qn_bf.shape[0]

    @pl.when(k == 0)
    def _():
        vs[...] = jnp.full(vs.shape, NEG, jnp.float32)
        is_[...] = jnp.full(is_.shape, IBIG, jnp.int32)
        q_blk = q_ref[...]
        q2 = jnp.sum(q_blk * q_blk, axis=1, keepdims=True)
        qn = q_blk / (jnp.sqrt(q2) + 1e-6)
        qn_bf[...] = qn.astype(jnp.bfloat16)

    m = m_ref[...]
    ss = jnp.sum(m * m, axis=1, keepdims=True)
    mn = (m / (jnp.sqrt(ss) + 1e-6)).astype(jnp.bfloat16)
    sims = lax.dot_general(mn, qn_bf[...], (((1,), (1,)), ((), ())),
                           preferred_element_type=jnp.float32)
    cb = k * kt + lax.broadcasted_iota(jnp.int32, (kt, 1), 0)
    work[...] = sims

    @pl.when(k == nk - 1)
    def _():
        if nk * kt != kreal:
            work[...] = jnp.where(cb < kreal, work[...], NEG)

    mxv[0:1, :] = jnp.max(work[...], axis=0, keepdims=True)

    def _cond(go):
        return go

    def _body(_):
        mx = mxv[0:1, :]
        t5 = vs[4:5, :]
        upd = mx > t5
        go = jnp.any(upd)

        @pl.when(go)
        def _():
            w = work[...]
            sel = jnp.min(jnp.where(w == mx, cb, IBIG), axis=0,
                          keepdims=True)
            v8 = vs[...]
            i8 = is_[...]
            ge = v8 >= mx
            gef = ge.astype(jnp.float32)
            gesh = jnp.concatenate(
                [jnp.ones((1, n_q), jnp.float32), gef[:7]], axis=0) > 0.5
            vsh = jnp.concatenate([v8[:1], v8[:7]], axis=0)
            ish = jnp.concatenate([i8[:1], i8[:7]], axis=0)
            nv = jnp.where(ge, v8, jnp.where(gesh, mx, vsh))
            ni = jnp.where(ge, i8, jnp.where(gesh, sel, ish))
            vs[...] = jnp.where(upd, nv, v8)
            is_[...] = jnp.where(upd, ni, i8)
            w2 = jnp.where(cb == sel, NEG, w)
            work[...] = w2
            mxv[0:1, :] = jnp.max(w2, axis=0, keepdims=True)

        return go

    lax.while_loop(_cond, _body, True)

    @pl.when(k == nk - 1)
    def _():
        vals_out[...] = vs[...]
        idx_out[...] = is_[...]


def _topk_pallas(q, memory, *, interpret=False):
    qn, d = q.shape
    kreal = memory.shape[0]
    kt = 1024
    nk = pl.cdiv(kreal, kt)
    body = functools.partial(_topk_body, kt=kt, nk=nk, kreal=kreal)
    return pl.pallas_call(
        body,
        grid=(nk,),
        in_specs=[
            pl.BlockSpec((qn, d), lambda k: (0, 0)),
            pl.BlockSpec((kt, d), lambda k: (k, 0)),
        ],
        out_specs=[
            pl.BlockSpec((8, qn), lambda k: (0, 0)),
            pl.BlockSpec((8, qn), lambda k: (0, 0)),
        ],
        out_shape=[
            jax.ShapeDtypeStruct((8, qn), jnp.float32),
            jax.ShapeDtypeStruct((8, qn), jnp.int32),
        ],
        scratch_shapes=[
            pltpu.VMEM((8, qn), jnp.float32),
            pltpu.VMEM((8, qn), jnp.int32),
            pltpu.VMEM((qn, d), jnp.bfloat16),
            pltpu.VMEM((kt, qn), jnp.float32),
            pltpu.VMEM((8, qn), jnp.float32),
        ],
        compiler_params=pltpu.CompilerParams(
            dimension_semantics=("arbitrary",),
        ),
        interpret=interpret,
    )(q, memory)


def _sc_gather(memory, idx3):
    from jax.experimental.pallas import tpu_sc as plsc

    nw, n_chunks, cw = idx3.shape
    d = memory.shape[1]
    n_rows = nw * n_chunks * cw
    mesh = plsc.VectorSubcoreMesh(core_axis_name="c", subcore_axis_name="s")
    info = plsc.get_sparse_core_info()
    nc = info.num_cores

    @functools.partial(
        pl.kernel,
        mesh=mesh,
        out_type=jax.ShapeDtypeStruct((n_rows, d), jnp.float32),
        scratch_types=[
            pltpu.VMEM((n_chunks, cw), jnp.int32),
            pltpu.VMEM((cw, d), jnp.float32),
            pltpu.SemaphoreType.DMA,
        ],
    )
    def gather_k(mem_hbm, idx_hbm, out_hbm, idx_v, rows_v, sem):
        wid = lax.axis_index("s") * nc + lax.axis_index("c")
        pltpu.sync_copy(idx_hbm.at[wid], idx_v)
        base = wid * (n_chunks * cw)
        for ch in range(n_chunks):
            pltpu.async_copy(mem_hbm.at[idx_v.at[ch]], rows_v, sem).wait()
            pltpu.sync_copy(rows_v, out_hbm.at[pl.ds(base + ch * cw, cw)])

    return gather_k(memory, idx3)


def _final_body(x_ref, neigh_ref, vals_ref, scores_ref, out_ref, loss_ref,
                *, n_tok):
    i = pl.program_id(0)
    x = x_ref[...]
    q2 = jnp.sum(x * x, axis=-1, keepdims=True)
    vals = vals_ref[...]
    w = jax.nn.softmax(vals, axis=-1)
    neigh = neigh_ref[...]
    retrieved = jnp.sum(w[..., None] * neigh, axis=2)
    mem_emb = 0.1 * x + 0.9 * retrieved
    out_ref[...] = mem_emb
    num = jnp.sum(x * mem_emb, axis=-1)
    den = (jnp.sqrt(q2[..., 0])
           * jnp.sqrt(jnp.sum(mem_emb * mem_emb, axis=-1)) + 1e-6)
    cos = num / den
    s = scores_ref[0]
    sn = s / (jnp.sum(s, axis=1, keepdims=True) + 1e-6)
    contrib = jnp.sum((1.0 - cos) * sn) / n_tok

    @pl.when(i == 0)
    def _():
        loss_ref[0, 0] = contrib

    @pl.when(i != 0)
    def _():
        loss_ref[0, 0] = loss_ref[0, 0] + contrib


def _final_pallas(x, neigh, vals8, scores, *, interpret=False):
    b, n, d = x.shape
    bb = 2
    grid = (b // bb,)
    body = functools.partial(_final_body, n_tok=float(b * n))
    return pl.pallas_call(
        body,
        grid=grid,
        in_specs=[
            pl.BlockSpec((bb, n, d), lambda i: (i, 0, 0)),
            pl.BlockSpec((bb, n, 5, d), lambda i: (i, 0, 0, 0)),
            pl.BlockSpec((bb, n, 5), lambda i: (i, 0, 0)),
            pl.BlockSpec((1, bb, n), lambda i: (i, 0, 0)),
        ],
        out_specs=[
            pl.BlockSpec((bb, n, d), lambda i: (i, 0, 0)),
            pl.BlockSpec((1, 1), lambda i: (0, 0),
                         memory_space=pltpu.SMEM),
        ],
        out_shape=[
            jax.ShapeDtypeStruct((b, n, d), jnp.float32),
            jax.ShapeDtypeStruct((1, 1), jnp.float32),
        ],
        compiler_params=pltpu.CompilerParams(
            dimension_semantics=("arbitrary",),
        ),
        interpret=interpret,
    )(x, neigh, vals8, scores.reshape(b // bb, bb, n))


def _run(x, memory, combined_scores, *, interpret=False, gather_fn=None):
    b, n, d = x.shape
    q = x.reshape(b * n, d)
    nq = b * n
    vals8t, idx8t = _topk_pallas(q, memory, interpret=interpret)
    idx_flat = idx8t[:5].T.reshape(nq * 5)
    n_chunks = pl.cdiv(nq * 5, 32 * 128)
    n_pad = 32 * n_chunks * 128
    idx_pad = jnp.concatenate(
        [idx_flat, jnp.zeros((n_pad - nq * 5,), jnp.int32)])
    idx3 = idx_pad.reshape(32, n_chunks, 128)
    if gather_fn is None:
        neigh_flat = _sc_gather(memory, idx3)
    else:
        neigh_flat = gather_fn(memory, idx3)
    neigh = neigh_flat[:nq * 5].reshape(b, n, 5, d)
    vals5 = vals8t[:5].T.reshape(b, n, 5)
    mem_emb, loss = _final_pallas(x, neigh, vals5, combined_scores,
                                  interpret=interpret)
    return mem_emb, loss[0, 0]


def kernel(x, memory, combined_scores, num_neighbors):
    del num_neighbors
    return _run(x, memory, combined_scores)


# speedup vs baseline: 1.0271x; 1.0167x over previous
import functools

import jax
import jax.numpy as jnp
from jax import lax
from jax.experimental import pallas as pl
from jax.experimental.pallas import tpu as pltpu

NEG = float("-inf")
IBIG = 2 ** 30


def _topk_body(q_ref, m_ref, vals_out, idx_out, vs, is_, qn_bf, work, mxv,
               *, kt, nk, kreal):
    k = pl.program_id(0)
    n_q = qn_bf.shape[0]

    @pl.when(k == 0)
    def _():
        vs[...] = jnp.full(vs.shape, NEG, jnp.float32)
        is_[...] = jnp.full(is_.shape, IBIG, jnp.int32)
        q_blk = q_ref[...]
        q2 = jnp.sum(q_blk * q_blk, axis=1, keepdims=True)
        qn = q_blk / (jnp.sqrt(q2) + 1e-6)
        qn_bf[...] = qn.astype(jnp.bfloat16)

    m = m_ref[...]
    ss = jnp.sum(m * m, axis=1, keepdims=True)
    mn = (m / (jnp.sqrt(ss) + 1e-6)).astype(jnp.bfloat16)
    sims = lax.dot_general(mn, qn_bf[...], (((1,), (1,)), ((), ())),
                           preferred_element_type=jnp.float32)
    cb = k * kt + lax.broadcasted_iota(jnp.int32, (kt, 1), 0)
    work[...] = sims

    @pl.when(k == nk - 1)
    def _():
        if nk * kt != kreal:
            work[...] = jnp.where(cb < kreal, work[...], NEG)

    mxv[0:1, :] = jnp.max(work[...], axis=0, keepdims=True)

    def _cond(go):
        return go

    def _body(_):
        mx = mxv[0:1, :]
        t5 = vs[4:5, :]
        upd = mx > t5
        go = jnp.any(upd)

        @pl.when(go)
        def _():
            w = work[...]
            sel = jnp.min(jnp.where(w == mx, cb, IBIG), axis=0,
                          keepdims=True)
            v8 = vs[...]
            i8 = is_[...]
            ge = v8 >= mx
            gef = ge.astype(jnp.float32)
            gesh = jnp.concatenate(
                [jnp.ones((1, n_q), jnp.float32), gef[:7]], axis=0) > 0.5
            vsh = jnp.concatenate([v8[:1], v8[:7]], axis=0)
            ish = jnp.concatenate([i8[:1], i8[:7]], axis=0)
            nv = jnp.where(ge, v8, jnp.where(gesh, mx, vsh))
            ni = jnp.where(ge, i8, jnp.where(gesh, sel, ish))
            vs[...] = jnp.where(upd, nv, v8)
            is_[...] = jnp.where(upd, ni, i8)
            w2 = jnp.where(cb == sel, NEG, w)
            work[...] = w2
            mxv[0:1, :] = jnp.max(w2, axis=0, keepdims=True)

        return go

    lax.while_loop(_cond, _body, True)

    @pl.when(k == nk - 1)
    def _():
        vals_out[...] = vs[...]
        idx_out[...] = is_[...]


def _topk_pallas(q, memory, *, interpret=False):
    qn, d = q.shape
    kreal = memory.shape[0]
    kt = 512
    nk = pl.cdiv(kreal, kt)
    body = functools.partial(_topk_body, kt=kt, nk=nk, kreal=kreal)
    return pl.pallas_call(
        body,
        grid=(nk,),
        in_specs=[
            pl.BlockSpec((qn, d), lambda k: (0, 0)),
            pl.BlockSpec((kt, d), lambda k: (k, 0)),
        ],
        out_specs=[
            pl.BlockSpec((8, qn), lambda k: (0, 0)),
            pl.BlockSpec((8, qn), lambda k: (0, 0)),
        ],
        out_shape=[
            jax.ShapeDtypeStruct((8, qn), jnp.float32),
            jax.ShapeDtypeStruct((8, qn), jnp.int32),
        ],
        scratch_shapes=[
            pltpu.VMEM((8, qn), jnp.float32),
            pltpu.VMEM((8, qn), jnp.int32),
            pltpu.VMEM((qn, d), jnp.bfloat16),
            pltpu.VMEM((kt, qn), jnp.float32),
            pltpu.VMEM((8, qn), jnp.float32),
        ],
        compiler_params=pltpu.CompilerParams(
            dimension_semantics=("arbitrary",),
        ),
        interpret=interpret,
    )(q, memory)


def _sc_gather(memory, idx3):
    from jax.experimental.pallas import tpu_sc as plsc

    nw, n_chunks, cw = idx3.shape
    d = memory.shape[1]
    n_rows = nw * n_chunks * cw
    mesh = plsc.VectorSubcoreMesh(core_axis_name="c", subcore_axis_name="s")
    info = plsc.get_sparse_core_info()
    nc = info.num_cores

    @functools.partial(
        pl.kernel,
        mesh=mesh,
        out_type=jax.ShapeDtypeStruct((n_rows, d), jnp.float32),
        scratch_types=[
            pltpu.VMEM((n_chunks, cw), jnp.int32),
            pltpu.VMEM((cw, d), jnp.float32),
            pltpu.SemaphoreType.DMA,
        ],
    )
    def gather_k(mem_hbm, idx_hbm, out_hbm, idx_v, rows_v, sem):
        wid = lax.axis_index("s") * nc + lax.axis_index("c")
        pltpu.sync_copy(idx_hbm.at[wid], idx_v)
        base = wid * (n_chunks * cw)
        for ch in range(n_chunks):
            pltpu.async_copy(mem_hbm.at[idx_v.at[ch]], rows_v, sem).wait()
            pltpu.sync_copy(rows_v, out_hbm.at[pl.ds(base + ch * cw, cw)])

    return gather_k(memory, idx3)


def _final_body(x_ref, neigh_ref, vals_ref, scores_ref, out_ref, loss_ref,
                *, n_tok):
    i = pl.program_id(0)
    x = x_ref[...]
    q2 = jnp.sum(x * x, axis=-1, keepdims=True)
    vals = vals_ref[...]
    w = jax.nn.softmax(vals, axis=-1)
    neigh = neigh_ref[...]
    retrieved = jnp.sum(w[..., None] * neigh, axis=2)
    mem_emb = 0.1 * x + 0.9 * retrieved
    out_ref[...] = mem_emb
    num = jnp.sum(x * mem_emb, axis=-1)
    den = (jnp.sqrt(q2[..., 0])
           * jnp.sqrt(jnp.sum(mem_emb * mem_emb, axis=-1)) + 1e-6)
    cos = num / den
    s = scores_ref[0]
    sn = s / (jnp.sum(s, axis=1, keepdims=True) + 1e-6)
    contrib = jnp.sum((1.0 - cos) * sn) / n_tok

    @pl.when(i == 0)
    def _():
        loss_ref[0, 0] = contrib

    @pl.when(i != 0)
    def _():
        loss_ref[0, 0] = loss_ref[0, 0] + contrib


def _final_pallas(x, neigh, vals8, scores, *, interpret=False):
    b, n, d = x.shape
    bb = 2
    grid = (b // bb,)
    body = functools.partial(_final_body, n_tok=float(b * n))
    return pl.pallas_call(
        body,
        grid=grid,
        in_specs=[
            pl.BlockSpec((bb, n, d), lambda i: (i, 0, 0)),
            pl.BlockSpec((bb, n, 5, d), lambda i: (i, 0, 0, 0)),
            pl.BlockSpec((bb, n, 5), lambda i: (i, 0, 0)),
            pl.BlockSpec((1, bb, n), lambda i: (i, 0, 0)),
        ],
        out_specs=[
            pl.BlockSpec((bb, n, d), lambda i: (i, 0, 0)),
            pl.BlockSpec((1, 1), lambda i: (0, 0),
                         memory_space=pltpu.SMEM),
        ],
        out_shape=[
            jax.ShapeDtypeStruct((b, n, d), jnp.float32),
            jax.ShapeDtypeStruct((1, 1), jnp.float32),
        ],
        compiler_params=pltpu.CompilerParams(
            dimension_semantics=("arbitrary",),
        ),
        interpret=interpret,
    )(x, neigh, vals8, scores.reshape(b // bb, bb, n))


def _run(x, memory, combined_scores, *, interpret=False, gather_fn=None):
    b, n, d = x.shape
    q = x.reshape(b * n, d)
    nq = b * n
    vals8t, idx8t = _topk_pallas(q, memory, interpret=interpret)
    idx_flat = idx8t[:5].T.reshape(nq * 5)
    n_chunks = pl.cdiv(nq * 5, 32 * 128)
    n_pad = 32 * n_chunks * 128
    idx_pad = jnp.concatenate(
        [idx_flat, jnp.zeros((n_pad - nq * 5,), jnp.int32)])
    idx3 = idx_pad.reshape(32, n_chunks, 128)
    if gather_fn is None:
        neigh_flat = _sc_gather(memory, idx3)
    else:
        neigh_flat = gather_fn(memory, idx3)
    neigh = neigh_flat[:nq * 5].reshape(b, n, 5, d)
    vals5 = vals8t[:5].T.reshape(b, n, 5)
    mem_emb, loss = _final_pallas(x, neigh, vals5, combined_scores,
                                  interpret=interpret)
    return mem_emb, loss[0, 0]


def kernel(x, memory, combined_scores, num_neighbors):
    del num_neighbors
    return _run(x, memory, combined_scores)


# speedup vs baseline: 1.0662x; 1.0381x over previous
import functools

import jax
import jax.numpy as jnp
from jax import lax
from jax.experimental import pallas as pl
from jax.experimental.pallas import tpu as pltpu

NEG = float("-inf")
IBIG = 2 ** 30


def _topk_body(q_ref, m_ref, vals_out, idx_out, vs, is_, qn_bf, work, mxv,
               *, kt, nk, kreal):
    k = pl.program_id(0)
    n_q = qn_bf.shape[0]

    @pl.when(k == 0)
    def _():
        vs[...] = jnp.full(vs.shape, NEG, jnp.float32)
        is_[...] = jnp.full(is_.shape, IBIG, jnp.int32)
        q_blk = q_ref[...]
        q2 = jnp.sum(q_blk * q_blk, axis=1, keepdims=True)
        qn = q_blk / (jnp.sqrt(q2) + 1e-6)
        qn_bf[...] = qn.astype(jnp.bfloat16)

    m = m_ref[...]
    ss = jnp.sum(m * m, axis=1, keepdims=True)
    mn = (m / (jnp.sqrt(ss) + 1e-6)).astype(jnp.bfloat16)
    sims = lax.dot_general(mn, qn_bf[...], (((1,), (1,)), ((), ())),
                           preferred_element_type=jnp.float32)
    cb = k * kt + lax.broadcasted_iota(jnp.int32, (kt, 1), 0)
    work[...] = sims
    mxv[0:1, :] = jnp.max(sims, axis=0, keepdims=True)

    @pl.when(k == nk - 1)
    def _():
        if nk * kt != kreal:
            w = jnp.where(cb < kreal, work[...], NEG)
            work[...] = w
            mxv[0:1, :] = jnp.max(w, axis=0, keepdims=True)

    def _cond(go):
        return go

    def _body(_):
        mx = mxv[0:1, :]
        t5 = vs[4:5, :]
        upd = mx > t5
        go = jnp.any(upd)

        @pl.when(go)
        def _():
            w = work[...]
            sel = jnp.min(jnp.where(w == mx, cb, IBIG), axis=0,
                          keepdims=True)
            v8 = vs[...]
            i8 = is_[...]
            ge = v8 >= mx
            gef = ge.astype(jnp.float32)
            gesh = jnp.concatenate(
                [jnp.ones((1, n_q), jnp.float32), gef[:7]], axis=0) > 0.5
            vsh = jnp.concatenate([v8[:1], v8[:7]], axis=0)
            ish = jnp.concatenate([i8[:1], i8[:7]], axis=0)
            nv = jnp.where(ge, v8, jnp.where(gesh, mx, vsh))
            ni = jnp.where(ge, i8, jnp.where(gesh, sel, ish))
            vs[...] = jnp.where(upd, nv, v8)
            is_[...] = jnp.where(upd, ni, i8)
            w2 = jnp.where(cb == sel, NEG, w)
            work[...] = w2
            mxv[0:1, :] = jnp.max(w2, axis=0, keepdims=True)

        return go

    lax.while_loop(_cond, _body, True)

    @pl.when(k == nk - 1)
    def _():
        vals_out[...] = vs[...]
        idx_out[...] = is_[...]


def _topk_pallas(q, memory, *, interpret=False):
    qn, d = q.shape
    kreal = memory.shape[0]
    kt = 512
    nk = pl.cdiv(kreal, kt)
    body = functools.partial(_topk_body, kt=kt, nk=nk, kreal=kreal)
    return pl.pallas_call(
        body,
        grid=(nk,),
        in_specs=[
            pl.BlockSpec((qn, d), lambda k: (0, 0)),
            pl.BlockSpec((kt, d), lambda k: (k, 0)),
        ],
        out_specs=[
            pl.BlockSpec((8, qn), lambda k: (0, 0)),
            pl.BlockSpec((8, qn), lambda k: (0, 0)),
        ],
        out_shape=[
            jax.ShapeDtypeStruct((8, qn), jnp.float32),
            jax.ShapeDtypeStruct((8, qn), jnp.int32),
        ],
        scratch_shapes=[
            pltpu.VMEM((8, qn), jnp.float32),
            pltpu.VMEM((8, qn), jnp.int32),
            pltpu.VMEM((qn, d), jnp.bfloat16),
            pltpu.VMEM((kt, qn), jnp.float32),
            pltpu.VMEM((8, qn), jnp.float32),
        ],
        compiler_params=pltpu.CompilerParams(
            dimension_semantics=("arbitrary",),
        ),
        interpret=interpret,
    )(q, memory)


def _sc_gather(memory, idx3):
    from jax.experimental.pallas import tpu_sc as plsc

    nw, n_chunks, cw = idx3.shape
    d = memory.shape[1]
    n_rows = nw * n_chunks * cw
    mesh = plsc.VectorSubcoreMesh(core_axis_name="c", subcore_axis_name="s")
    info = plsc.get_sparse_core_info()
    nc = info.num_cores

    @functools.partial(
        pl.kernel,
        mesh=mesh,
        out_type=jax.ShapeDtypeStruct((n_rows, d), jnp.float32),
        scratch_types=[
            pltpu.VMEM((n_chunks, cw), jnp.int32),
            pltpu.VMEM((cw, d), jnp.float32),
            pltpu.SemaphoreType.DMA,
        ],
    )
    def gather_k(mem_hbm, idx_hbm, out_hbm, idx_v, rows_v, sem):
        wid = lax.axis_index("s") * nc + lax.axis_index("c")
        pltpu.sync_copy(idx_hbm.at[wid], idx_v)
        base = wid * (n_chunks * cw)
        for ch in range(n_chunks):
            pltpu.async_copy(mem_hbm.at[idx_v.at[ch]], rows_v, sem).wait()
            pltpu.sync_copy(rows_v, out_hbm.at[pl.ds(base + ch * cw, cw)])

    return gather_k(memory, idx3)


def _final_body(x_ref, neigh_ref, vals_ref, scores_ref, out_ref, loss_ref,
                *, n_tok):
    i = pl.program_id(0)
    x = x_ref[...]
    q2 = jnp.sum(x * x, axis=-1, keepdims=True)
    vals = vals_ref[...]
    w = jax.nn.softmax(vals, axis=-1)
    neigh = neigh_ref[...]
    retrieved = jnp.sum(w[..., None] * neigh, axis=2)
    mem_emb = 0.1 * x + 0.9 * retrieved
    out_ref[...] = mem_emb
    num = jnp.sum(x * mem_emb, axis=-1)
    den = (jnp.sqrt(q2[..., 0])
           * jnp.sqrt(jnp.sum(mem_emb * mem_emb, axis=-1)) + 1e-6)
    cos = num / den
    s = scores_ref[0]
    sn = s / (jnp.sum(s, axis=1, keepdims=True) + 1e-6)
    contrib = jnp.sum((1.0 - cos) * sn) / n_tok

    @pl.when(i == 0)
    def _():
        loss_ref[0, 0] = contrib

    @pl.when(i != 0)
    def _():
        loss_ref[0, 0] = loss_ref[0, 0] + contrib


def _final_pallas(x, neigh, vals8, scores, *, interpret=False):
    b, n, d = x.shape
    bb = 2
    grid = (b // bb,)
    body = functools.partial(_final_body, n_tok=float(b * n))
    return pl.pallas_call(
        body,
        grid=grid,
        in_specs=[
            pl.BlockSpec((bb, n, d), lambda i: (i, 0, 0)),
            pl.BlockSpec((bb, n, 5, d), lambda i: (i, 0, 0, 0)),
            pl.BlockSpec((bb, n, 5), lambda i: (i, 0, 0)),
            pl.BlockSpec((1, bb, n), lambda i: (i, 0, 0)),
        ],
        out_specs=[
            pl.BlockSpec((bb, n, d), lambda i: (i, 0, 0)),
            pl.BlockSpec((1, 1), lambda i: (0, 0),
                         memory_space=pltpu.SMEM),
        ],
        out_shape=[
            jax.ShapeDtypeStruct((b, n, d), jnp.float32),
            jax.ShapeDtypeStruct((1, 1), jnp.float32),
        ],
        compiler_params=pltpu.CompilerParams(
            dimension_semantics=("arbitrary",),
        ),
        interpret=interpret,
    )(x, neigh, vals8, scores.reshape(b // bb, bb, n))


def _run(x, memory, combined_scores, *, interpret=False, gather_fn=None):
    b, n, d = x.shape
    q = x.reshape(b * n, d)
    nq = b * n
    vals8t, idx8t = _topk_pallas(q, memory, interpret=interpret)
    idx_flat = idx8t[:5].T.reshape(nq * 5)
    n_chunks = pl.cdiv(nq * 5, 32 * 128)
    n_pad = 32 * n_chunks * 128
    idx_pad = jnp.concatenate(
        [idx_flat, jnp.zeros((n_pad - nq * 5,), jnp.int32)])
    idx3 = idx_pad.reshape(32, n_chunks, 128)
    if gather_fn is None:
        neigh_flat = _sc_gather(memory, idx3)
    else:
        neigh_flat = gather_fn(memory, idx3)
    neigh = neigh_flat[:nq * 5].reshape(b, n, 5, d)
    vals5 = vals8t[:5].T.reshape(b, n, 5)
    mem_emb, loss = _final_pallas(x, neigh, vals5, combined_scores,
                                  interpret=interpret)
    return mem_emb, loss[0, 0]


def kernel(x, memory, combined_scores, num_neighbors):
    del num_neighbors
    return _run(x, memory, combined_scores)


# speedup vs baseline: 1.0822x; 1.0150x over previous
import functools

import jax
import jax.numpy as jnp
from jax import lax
from jax.experimental import pallas as pl
from jax.experimental.pallas import tpu as pltpu

NEG = float("-inf")
IBIG = 2 ** 30


def _topk_body(q_ref, m_ref, vals_out, idx_out, vs, is_, qn_bf, work, mxv,
               *, kt, nk, kreal):
    k = pl.program_id(0)
    n_q = qn_bf.shape[0]

    @pl.when(k == 0)
    def _():
        vs[...] = jnp.full(vs.shape, NEG, jnp.float32)
        is_[...] = jnp.full(is_.shape, IBIG, jnp.int32)
        q_blk = q_ref[...]
        q2 = jnp.sum(q_blk * q_blk, axis=1, keepdims=True)
        qn = q_blk / (jnp.sqrt(q2) + 1e-6)
        qn_bf[...] = qn.astype(jnp.bfloat16)

    m = m_ref[...]
    ss = jnp.sum(m * m, axis=1, keepdims=True)
    mn = (m / (jnp.sqrt(ss) + 1e-6)).astype(jnp.bfloat16)
    sims = lax.dot_general(mn, qn_bf[...], (((1,), (1,)), ((), ())),
                           preferred_element_type=jnp.float32)
    half = kt // 2
    cb0 = lax.broadcasted_iota(jnp.int32, (half, 1), 0)
    work[...] = sims
    mxv[0:1, :] = jnp.max(sims[:half], axis=0, keepdims=True)
    mxv[1:2, :] = jnp.max(sims[half:], axis=0, keepdims=True)

    @pl.when(k == nk - 1)
    def _():
        if nk * kt != kreal:
            for h in range(2):
                cb = k * kt + h * half + cb0
                w = jnp.where(cb < kreal, work[h * half:(h + 1) * half, :],
                              NEG)
                work[h * half:(h + 1) * half, :] = w
                mxv[h:h + 1, :] = jnp.max(w, axis=0, keepdims=True)

    def _cond(go):
        return go

    def _make_body(h):
        lo, hi = h * half, (h + 1) * half
        cb = k * kt + h * half + cb0

        def _body(_):
            mx = mxv[h:h + 1, :]
            t5 = vs[4:5, :]
            upd = mx > t5
            go = jnp.any(upd)

            @pl.when(go)
            def _():
                w = work[lo:hi, :]
                sel = jnp.min(jnp.where(w == mx, cb, IBIG), axis=0,
                              keepdims=True)
                v8 = vs[...]
                i8 = is_[...]
                ge = v8 >= mx
                gef = ge.astype(jnp.float32)
                gesh = jnp.concatenate(
                    [jnp.ones((1, n_q), jnp.float32), gef[:7]],
                    axis=0) > 0.5
                vsh = jnp.concatenate([v8[:1], v8[:7]], axis=0)
                ish = jnp.concatenate([i8[:1], i8[:7]], axis=0)
                nv = jnp.where(ge, v8, jnp.where(gesh, mx, vsh))
                ni = jnp.where(ge, i8, jnp.where(gesh, sel, ish))
                vs[...] = jnp.where(upd, nv, v8)
                is_[...] = jnp.where(upd, ni, i8)
                w2 = jnp.where(cb == sel, NEG, w)
                work[lo:hi, :] = w2
                mxv[h:h + 1, :] = jnp.max(w2, axis=0, keepdims=True)

            return go

        return _body

    lax.while_loop(_cond, _make_body(0), True)
    lax.while_loop(_cond, _make_body(1), True)

    @pl.when(k == nk - 1)
    def _():
        vals_out[...] = vs[...]
        idx_out[...] = is_[...]


def _topk_pallas(q, memory, *, interpret=False):
    qn, d = q.shape
    kreal = memory.shape[0]
    kt = 512
    nk = pl.cdiv(kreal, kt)
    body = functools.partial(_topk_body, kt=kt, nk=nk, kreal=kreal)
    return pl.pallas_call(
        body,
        grid=(nk,),
        in_specs=[
            pl.BlockSpec((qn, d), lambda k: (0, 0)),
            pl.BlockSpec((kt, d), lambda k: (k, 0)),
        ],
        out_specs=[
            pl.BlockSpec((8, qn), lambda k: (0, 0)),
            pl.BlockSpec((8, qn), lambda k: (0, 0)),
        ],
        out_shape=[
            jax.ShapeDtypeStruct((8, qn), jnp.float32),
            jax.ShapeDtypeStruct((8, qn), jnp.int32),
        ],
        scratch_shapes=[
            pltpu.VMEM((8, qn), jnp.float32),
            pltpu.VMEM((8, qn), jnp.int32),
            pltpu.VMEM((qn, d), jnp.bfloat16),
            pltpu.VMEM((kt, qn), jnp.float32),
            pltpu.VMEM((8, qn), jnp.float32),
        ],
        compiler_params=pltpu.CompilerParams(
            dimension_semantics=("arbitrary",),
        ),
        interpret=interpret,
    )(q, memory)


def _sc_gather(memory, idx3):
    from jax.experimental.pallas import tpu_sc as plsc

    nw, n_chunks, cw = idx3.shape
    d = memory.shape[1]
    n_rows = nw * n_chunks * cw
    mesh = plsc.VectorSubcoreMesh(core_axis_name="c", subcore_axis_name="s")
    info = plsc.get_sparse_core_info()
    nc = info.num_cores

    @functools.partial(
        pl.kernel,
        mesh=mesh,
        out_type=jax.ShapeDtypeStruct((n_rows, d), jnp.float32),
        scratch_types=[
            pltpu.VMEM((n_chunks, cw), jnp.int32),
            pltpu.VMEM((cw, d), jnp.float32),
            pltpu.SemaphoreType.DMA,
        ],
    )
    def gather_k(mem_hbm, idx_hbm, out_hbm, idx_v, rows_v, sem):
        wid = lax.axis_index("s") * nc + lax.axis_index("c")
        pltpu.sync_copy(idx_hbm.at[wid], idx_v)
        base = wid * (n_chunks * cw)
        for ch in range(n_chunks):
            pltpu.async_copy(mem_hbm.at[idx_v.at[ch]], rows_v, sem).wait()
            pltpu.sync_copy(rows_v, out_hbm.at[pl.ds(base + ch * cw, cw)])

    return gather_k(memory, idx3)


def _final_body(x_ref, neigh_ref, vals_ref, scores_ref, out_ref, loss_ref,
                *, n_tok):
    i = pl.program_id(0)
    x = x_ref[...]
    q2 = jnp.sum(x * x, axis=-1, keepdims=True)
    vals = vals_ref[...]
    w = jax.nn.softmax(vals, axis=-1)
    neigh = neigh_ref[...]
    retrieved = jnp.sum(w[..., None] * neigh, axis=2)
    mem_emb = 0.1 * x + 0.9 * retrieved
    out_ref[...] = mem_emb
    num = jnp.sum(x * mem_emb, axis=-1)
    den = (jnp.sqrt(q2[..., 0])
           * jnp.sqrt(jnp.sum(mem_emb * mem_emb, axis=-1)) + 1e-6)
    cos = num / den
    s = scores_ref[0]
    sn = s / (jnp.sum(s, axis=1, keepdims=True) + 1e-6)
    contrib = jnp.sum((1.0 - cos) * sn) / n_tok

    @pl.when(i == 0)
    def _():
        loss_ref[0, 0] = contrib

    @pl.when(i != 0)
    def _():
        loss_ref[0, 0] = loss_ref[0, 0] + contrib


def _final_pallas(x, neigh, vals8, scores, *, interpret=False):
    b, n, d = x.shape
    bb = 2
    grid = (b // bb,)
    body = functools.partial(_final_body, n_tok=float(b * n))
    return pl.pallas_call(
        body,
        grid=grid,
        in_specs=[
            pl.BlockSpec((bb, n, d), lambda i: (i, 0, 0)),
            pl.BlockSpec((bb, n, 5, d), lambda i: (i, 0, 0, 0)),
            pl.BlockSpec((bb, n, 5), lambda i: (i, 0, 0)),
            pl.BlockSpec((1, bb, n), lambda i: (i, 0, 0)),
        ],
        out_specs=[
            pl.BlockSpec((bb, n, d), lambda i: (i, 0, 0)),
            pl.BlockSpec((1, 1), lambda i: (0, 0),
                         memory_space=pltpu.SMEM),
        ],
        out_shape=[
            jax.ShapeDtypeStruct((b, n, d), jnp.float32),
            jax.ShapeDtypeStruct((1, 1), jnp.float32),
        ],
        compiler_params=pltpu.CompilerParams(
            dimension_semantics=("arbitrary",),
        ),
        interpret=interpret,
    )(x, neigh, vals8, scores.reshape(b // bb, bb, n))


def _run(x, memory, combined_scores, *, interpret=False, gather_fn=None):
    b, n, d = x.shape
    q = x.reshape(b * n, d)
    nq = b * n
    vals8t, idx8t = _topk_pallas(q, memory, interpret=interpret)
    idx_flat = idx8t[:5].T.reshape(nq * 5)
    n_chunks = pl.cdiv(nq * 5, 32 * 128)
    n_pad = 32 * n_chunks * 128
    idx_pad = jnp.concatenate(
        [idx_flat, jnp.zeros((n_pad - nq * 5,), jnp.int32)])
    idx3 = idx_pad.reshape(32, n_chunks, 128)
    if gather_fn is None:
        neigh_flat = _sc_gather(memory, idx3)
    else:
        neigh_flat = gather_fn(memory, idx3)
    neigh = neigh_flat[:nq * 5].reshape(b, n, 5, d)
    vals5 = vals8t[:5].T.reshape(b, n, 5)
    mem_emb, loss = _final_pallas(x, neigh, vals5, combined_scores,
                                  interpret=interpret)
    return mem_emb, loss[0, 0]


def kernel(x, memory, combined_scores, num_neighbors):
    del num_neighbors
    return _run(x, memory, combined_scores)


# speedup vs baseline: 1.0836x; 1.0013x over previous
import functools

import jax
import jax.numpy as jnp
from jax import lax
from jax.experimental import pallas as pl
from jax.experimental.pallas import tpu as pltpu

NEG = float("-inf")
IBIG = 2 ** 30


def _topk_body(q_ref, m_ref, vals_out, idx_out, vs, is_, qn_bf, work, mxv,
               *, kt, nk, kreal):
    k = pl.program_id(0)
    n_q = qn_bf.shape[0]

    @pl.when(k == 0)
    def _():
        vs[...] = jnp.full(vs.shape, NEG, jnp.float32)
        is_[...] = jnp.full(is_.shape, IBIG, jnp.int32)
        q_blk = q_ref[...]
        q2 = jnp.sum(q_blk * q_blk, axis=1, keepdims=True)
        qn = q_blk / (jnp.sqrt(q2) + 1e-6)
        qn_bf[...] = qn.astype(jnp.bfloat16)

    m = m_ref[...]
    ss = jnp.sum(m * m, axis=1, keepdims=True)
    mn = (m / (jnp.sqrt(ss) + 1e-6)).astype(jnp.bfloat16)
    sims = lax.dot_general(mn, qn_bf[...], (((1,), (1,)), ((), ())),
                           preferred_element_type=jnp.float32)
    half = kt // 2
    cb0 = lax.broadcasted_iota(jnp.int32, (half, 1), 0)
    work[...] = sims
    mxv[0:1, :] = jnp.max(sims[:half], axis=0, keepdims=True)
    mxv[1:2, :] = jnp.max(sims[half:], axis=0, keepdims=True)

    @pl.when(k == nk - 1)
    def _():
        if nk * kt != kreal:
            for h in range(2):
                cb = k * kt + h * half + cb0
                w = jnp.where(cb < kreal, work[h * half:(h + 1) * half, :],
                              NEG)
                work[h * half:(h + 1) * half, :] = w
                mxv[h:h + 1, :] = jnp.max(w, axis=0, keepdims=True)

    def _cond(go):
        return go

    def _make_body(h):
        lo, hi = h * half, (h + 1) * half
        cb = k * kt + h * half + cb0

        def _body(_):
            mx = mxv[h:h + 1, :]
            t5 = vs[4:5, :]
            upd = mx > t5
            go = jnp.any(upd)

            @pl.when(go)
            def _():
                w = work[lo:hi, :]
                sel = jnp.min(jnp.where(w == mx, cb, IBIG), axis=0,
                              keepdims=True)
                v8 = vs[...]
                i8 = is_[...]
                ge = v8 >= mx
                gef = ge.astype(jnp.float32)
                gesh = jnp.concatenate(
                    [jnp.ones((1, n_q), jnp.float32), gef[:7]],
                    axis=0) > 0.5
                vsh = jnp.concatenate([v8[:1], v8[:7]], axis=0)
                ish = jnp.concatenate([i8[:1], i8[:7]], axis=0)
                nv = jnp.where(ge, v8, jnp.where(gesh, mx, vsh))
                ni = jnp.where(ge, i8, jnp.where(gesh, sel, ish))
                vs[...] = jnp.where(upd, nv, v8)
                is_[...] = jnp.where(upd, ni, i8)
                w2 = jnp.where(cb == sel, NEG, w)
                work[lo:hi, :] = w2
                mxv[h:h + 1, :] = jnp.max(w2, axis=0, keepdims=True)

            return go

        return _body

    lax.while_loop(_cond, _make_body(0), True)
    lax.while_loop(_cond, _make_body(1), True)

    @pl.when(k == nk - 1)
    def _():
        vals_out[...] = vs[...]
        idx_out[...] = is_[...]


def _topk_pallas(q, memory):
    qn, d = q.shape
    kreal = memory.shape[0]
    kt = 512
    nk = pl.cdiv(kreal, kt)
    body = functools.partial(_topk_body, kt=kt, nk=nk, kreal=kreal)
    return pl.pallas_call(
        body,
        grid=(nk,),
        in_specs=[
            pl.BlockSpec((qn, d), lambda k: (0, 0)),
            pl.BlockSpec((kt, d), lambda k: (k, 0)),
        ],
        out_specs=[
            pl.BlockSpec((8, qn), lambda k: (0, 0)),
            pl.BlockSpec((8, qn), lambda k: (0, 0)),
        ],
        out_shape=[
            jax.ShapeDtypeStruct((8, qn), jnp.float32),
            jax.ShapeDtypeStruct((8, qn), jnp.int32),
        ],
        scratch_shapes=[
            pltpu.VMEM((8, qn), jnp.float32),
            pltpu.VMEM((8, qn), jnp.int32),
            pltpu.VMEM((qn, d), jnp.bfloat16),
            pltpu.VMEM((kt, qn), jnp.float32),
            pltpu.VMEM((8, qn), jnp.float32),
        ],
        compiler_params=pltpu.CompilerParams(
            dimension_semantics=("arbitrary",),
        ),
    )(q, memory)


def _sc_gather(memory, idx3):
    from jax.experimental.pallas import tpu_sc as plsc

    nw, n_chunks, cw = idx3.shape
    d = memory.shape[1]
    n_rows = nw * n_chunks * cw
    mesh = plsc.VectorSubcoreMesh(core_axis_name="c", subcore_axis_name="s")
    info = plsc.get_sparse_core_info()
    nc = info.num_cores

    @functools.partial(
        pl.kernel,
        mesh=mesh,
        out_type=jax.ShapeDtypeStruct((n_rows, d), jnp.float32),
        scratch_types=[
            pltpu.VMEM((n_chunks, cw), jnp.int32),
            pltpu.VMEM((cw, d), jnp.float32),
            pltpu.SemaphoreType.DMA,
        ],
    )
    def gather_k(mem_hbm, idx_hbm, out_hbm, idx_v, rows_v, sem):
        wid = lax.axis_index("s") * nc + lax.axis_index("c")
        pltpu.sync_copy(idx_hbm.at[wid], idx_v)
        base = wid * (n_chunks * cw)
        for ch in range(n_chunks):
            pltpu.async_copy(mem_hbm.at[idx_v.at[ch]], rows_v, sem).wait()
            pltpu.sync_copy(rows_v, out_hbm.at[pl.ds(base + ch * cw, cw)])

    return gather_k(memory, idx3)


def _final_body(x_ref, neigh_ref, vals_ref, scores_ref, out_ref, loss_ref,
                *, n_tok):
    i = pl.program_id(0)
    x = x_ref[...]
    q2 = jnp.sum(x * x, axis=-1, keepdims=True)
    vals = vals_ref[...]
    w = jax.nn.softmax(vals, axis=-1)
    neigh = neigh_ref[...]
    retrieved = jnp.sum(w[..., None] * neigh, axis=2)
    mem_emb = 0.1 * x + 0.9 * retrieved
    out_ref[...] = mem_emb
    num = jnp.sum(x * mem_emb, axis=-1)
    den = (jnp.sqrt(q2[..., 0])
           * jnp.sqrt(jnp.sum(mem_emb * mem_emb, axis=-1)) + 1e-6)
    cos = num / den
    s = scores_ref[0]
    sn = s / (jnp.sum(s, axis=1, keepdims=True) + 1e-6)
    contrib = jnp.sum((1.0 - cos) * sn) / n_tok

    @pl.when(i == 0)
    def _():
        loss_ref[0, 0] = contrib

    @pl.when(i != 0)
    def _():
        loss_ref[0, 0] = loss_ref[0, 0] + contrib


def _final_pallas(x, neigh, vals5, scores):
    b, n, d = x.shape
    bb = 2
    grid = (b // bb,)
    body = functools.partial(_final_body, n_tok=float(b * n))
    return pl.pallas_call(
        body,
        grid=grid,
        in_specs=[
            pl.BlockSpec((bb, n, d), lambda i: (i, 0, 0)),
            pl.BlockSpec((bb, n, 5, d), lambda i: (i, 0, 0, 0)),
            pl.BlockSpec((bb, n, 5), lambda i: (i, 0, 0)),
            pl.BlockSpec((1, bb, n), lambda i: (i, 0, 0)),
        ],
        out_specs=[
            pl.BlockSpec((bb, n, d), lambda i: (i, 0, 0)),
            pl.BlockSpec((1, 1), lambda i: (0, 0),
                         memory_space=pltpu.SMEM),
        ],
        out_shape=[
            jax.ShapeDtypeStruct((b, n, d), jnp.float32),
            jax.ShapeDtypeStruct((1, 1), jnp.float32),
        ],
        compiler_params=pltpu.CompilerParams(
            dimension_semantics=("arbitrary",),
        ),
    )(x, neigh, vals5, scores.reshape(b // bb, bb, n))


def _run(x, memory, combined_scores):
    b, n, d = x.shape
    q = x.reshape(b * n, d)
    nq = b * n
    vals8t, idx8t = _topk_pallas(q, memory)
    idx_flat = idx8t[:5].T.reshape(nq * 5)
    n_chunks = pl.cdiv(nq * 5, 32 * 128)
    n_pad = 32 * n_chunks * 128
    idx_pad = jnp.concatenate(
        [idx_flat, jnp.zeros((n_pad - nq * 5,), jnp.int32)])
    idx3 = idx_pad.reshape(32, n_chunks, 128)
    neigh_flat = _sc_gather(memory, idx3)
    neigh = neigh_flat[:nq * 5].reshape(b, n, 5, d)
    vals5 = vals8t[:5].T.reshape(b, n, 5)
    mem_emb, loss = _final_pallas(x, neigh, vals5, combined_scores)
    return mem_emb, loss[0, 0]


def kernel(x, memory, combined_scores, num_neighbors):
    del num_neighbors
    return _run(x, memory, combined_scores)
